# baseline (speedup 1.0000x reference)
import jax
import jax.numpy as jnp
import numpy as np
from jax import lax
from jax.experimental import pallas as pl
from jax.experimental.pallas import tpu as pltpu
from jax.experimental.pallas import tpu_sc as plsc

N = 10000
E = 320000
DG = 512
W = 128
WW = W // 2
NCH = 2
NS = 16
B = 64
NBLK = 316
EPT = NBLK * B
EPAD = NS * EPT
NPAD = 10240
RPT = NPAD // NS

_PERM32 = np.stack([np.arange(16), np.arange(16) + 16], axis=1).reshape(32)


def _mm_body(x_ref, h_ref, wx_ref, wh_ref, o_ref):
    o_ref[0] = (
        jnp.dot(x_ref[...], wx_ref[...], preferred_element_type=jnp.float32)
        + jnp.dot(h_ref[...], wh_ref[...], preferred_element_type=jnp.float32)
    )


def _matmul(X, H, Wx, Wh):
    return pl.pallas_call(
        _mm_body,
        grid=(4, 10),
        in_specs=[
            pl.BlockSpec((1000, 128), lambda j, i: (i, 0)),
            pl.BlockSpec((1000, 128), lambda j, i: (i, 0)),
            pl.BlockSpec((128, 128), lambda j, i: (0, j)),
            pl.BlockSpec((128, 128), lambda j, i: (0, j)),
        ],
        out_specs=pl.BlockSpec((1, 1000, 128), lambda j, i: (j, i, 0)),
        out_shape=jax.ShapeDtypeStruct((4, N, 128), jnp.float32),
    )(X, H, Wx, Wh)


def _sc_body(ep_hbm, m_hbm, z_hbm,
             s_out, dis_out,
             ssp, degsp,
             ebuf0, ebuf1, idx0, idx1, ab0, ab1, cb0, cb1,
             dis_t, gbuf0, gbuf1, sbuf0, sbuf1, tmp_t,
             esem0, esem1, gsem0, gsem1, ssem0, ssem1):
    c = lax.axis_index("c")
    s = lax.axis_index("s")
    base = s * RPT

    ebufs = (ebuf0, ebuf1)
    idxs = (idx0, idx1)
    abs_ = (ab0, ab1)
    cbs = (cb0, cb1)
    gbufs = (gbuf0, gbuf1)
    sbufs = (sbuf0, sbuf1)
    esems = (esem0, esem1)
    gsems = (gsem0, gsem1)
    ssems = (ssem0, ssem1)
    NG = B // 16

    def estart(b, p):
        pltpu.async_copy(ep_hbm.at[s, b], ebufs[p], esems[p])

    def ewait(b, p):
        pltpu.make_async_copy(ep_hbm.at[s, b], ebufs[p], esems[p]).wait()

    def edrain():
        ewait(NBLK, 0)
        ewait(NBLK + 1, 1)

    def fill_ones(i, carry):
        tmp_t[pl.ds(i * 16, 16)] = jnp.ones((16,), jnp.float32)
        return carry
    lax.fori_loop(0, RPT // 16, fill_ones, 0)
    pltpu.sync_copy(tmp_t, degsp.at[pl.ds(base, RPT)])
    plsc.subcore_barrier()

    def deg_step(b, p, first):
        if not first:
            pltpu.make_async_copy(
                abs_[p], degsp.at[cbs[p]], ssems[p]).wait()
        ewait(b, p)

        def cvt(g, carry2):
            sl = pl.ds(g * 16, 16)
            abs_[p][sl] = lax.bitcast_convert_type(
                ebufs[p][pl.ds(2 * B + g * 16, 16)], jnp.float32)
            cbs[p][sl] = ebufs[p][pl.ds(B + g * 16, 16)]
            return carry2
        lax.fori_loop(0, NG, cvt, 0)
        estart(b + 2, p)
        pltpu.async_copy(abs_[p], degsp.at[cbs[p]], ssems[p], add=True)

    estart(0, 0)
    estart(1, 1)
    deg_step(0, 0, True)
    deg_step(1, 1, True)

    def deg_pipe(j, carry):
        deg_step(2 * j + 2, 0, False)
        deg_step(2 * j + 3, 1, False)
        return carry
    lax.fori_loop(0, NBLK // 2 - 1, deg_pipe, 0)
    for p in range(2):
        pltpu.make_async_copy(abs_[p], degsp.at[cbs[p]], ssems[p]).wait()
    edrain()
    plsc.subcore_barrier()

    pltpu.sync_copy(degsp.at[pl.ds(base, RPT)], tmp_t)

    def rsqrt_vec(i, carry):
        x = tmp_t[pl.ds(i * 16, 16)]
        ii = lax.bitcast_convert_type(x, jnp.int32)
        ii = jnp.int32(0x5F3759DF) - lax.shift_right_logical(ii, 1)
        y = lax.bitcast_convert_type(ii, jnp.float32)
        y = y * (1.5 - 0.5 * x * y * y)
        y = y * (1.5 - 0.5 * x * y * y)
        y = y * (1.5 - 0.5 * x * y * y)
        tmp_t[pl.ds(i * 16, 16)] = y
        return carry
    lax.fori_loop(0, RPT // 16, rsqrt_vec, 0)
    pltpu.sync_copy(tmp_t, degsp.at[pl.ds(base, RPT)])

    @pl.when(c == 0)
    def _():
        pltpu.sync_copy(tmp_t, dis_out.at[pl.ds(base, RPT)])
    plsc.subcore_barrier()

    pltpu.sync_copy(degsp, dis_t)

    def chunk(k, carry):
        ci = c * NCH + k

        pltpu.sync_copy(z_hbm.at[pl.ds(base, RPT)], ssp.at[pl.ds(base, RPT)])
        plsc.subcore_barrier()

        def issue(b, p, first):
            if not first:
                pltpu.make_async_copy(
                    sbufs[p], ssp.at[cbs[p]], ssems[p]).wait()
            ewait(b, p)

            def prep(g, carry3):
                sl = pl.ds(g * 16, 16)
                r16 = ebufs[p][sl]
                idxs[p][sl] = r16 + ci * N
                cbs[p][sl] = ebufs[p][pl.ds(B + g * 16, 16)]
                ew16 = lax.bitcast_convert_type(
                    ebufs[p][pl.ds(2 * B + g * 16, 16)], jnp.float32)
                abs_[p][sl] = ew16 * plsc.load_gather(dis_t, [r16])
                return carry3
            lax.fori_loop(0, NG, prep, 0)
            estart(b + 2, p)
            pltpu.async_copy(m_hbm.at[idxs[p]], gbufs[p], gsems[p])

        def retire(b, p):
            pltpu.make_async_copy(m_hbm.at[idxs[p]], gbufs[p], gsems[p]).wait()

            def grp(g, carry3):
                a16 = abs_[p][pl.ds(g * 16, 16)]
                for e in range(16):
                    ae = jnp.take_along_axis(
                        a16, jnp.full((16,), e, jnp.int32), axis=0)
                    aeb = plsc.pack(ae, ae,
                                    format=plsc.PackFormat.INTERLEAVED)
                    r = g * 16 + e
                    for q in range(WW // 16):
                        wv = gbufs[p][r, pl.ds(q * 16, 16)]
                        vb = plsc.bitcast(wv, jnp.bfloat16)
                        pr = vb * aeb
                        u0, u1 = plsc.unpack(
                            pr, format=plsc.PackFormat.INTERLEAVED)
                        sbufs[p][r, pl.ds(q * 32, 16)] = u0
                        sbufs[p][r, pl.ds(q * 32 + 16, 16)] = u1
                return carry3
            lax.fori_loop(0, NG, grp, 0)
            pltpu.async_copy(sbufs[p], ssp.at[cbs[p]], ssems[p], add=True)

        estart(0, 0)
        estart(1, 1)
        issue(0, 0, True)
        issue(1, 1, True)

        def pipe(j, carry2):
            retire(2 * j, 0)
            issue(2 * j + 2, 0, False)
            retire(2 * j + 1, 1)
            issue(2 * j + 3, 1, False)
            return carry2
        lax.fori_loop(0, NBLK // 2 - 1, pipe, 0)
        retire(NBLK - 2, 0)
        retire(NBLK - 1, 1)
        for p in range(2):
            pltpu.make_async_copy(
                sbufs[p], ssp.at[cbs[p]], ssems[p]).wait()
        edrain()

        plsc.subcore_barrier()
        pltpu.sync_copy(ssp.at[pl.ds(base, RPT)],
                        s_out.at[ci, pl.ds(base, RPT)])
        plsc.subcore_barrier()
        return carry
    lax.fori_loop(0, NCH, chunk, 0)


def _sc_call(epack, m_i32, zeros):
    mesh = plsc.VectorSubcoreMesh(core_axis_name="c", subcore_axis_name="s",
                                  num_cores=2, num_subcores=NS)
    return pl.kernel(
        _sc_body,
        out_type=(
            jax.ShapeDtypeStruct((2 * NCH, NPAD, W), jnp.float32),
            jax.ShapeDtypeStruct((NPAD,), jnp.float32),
        ),
        mesh=mesh,
        compiler_params=pltpu.CompilerParams(needs_layout_passes=False,
                                             use_tc_tiling_on_sc=False),
        scratch_types=[
            pltpu.VMEM_SHARED((NPAD, W), jnp.float32),
            pltpu.VMEM_SHARED((NPAD,), jnp.float32),
            pltpu.VMEM((3 * B,), jnp.int32),
            pltpu.VMEM((3 * B,), jnp.int32),
            pltpu.VMEM((B,), jnp.int32),
            pltpu.VMEM((B,), jnp.int32),
            pltpu.VMEM((B,), jnp.float32),
            pltpu.VMEM((B,), jnp.float32),
            pltpu.VMEM((B,), jnp.int32),
            pltpu.VMEM((B,), jnp.int32),
            pltpu.VMEM((NPAD,), jnp.float32),
            pltpu.VMEM((B, WW), jnp.int32),
            pltpu.VMEM((B, WW), jnp.int32),
            pltpu.VMEM((B, W), jnp.float32),
            pltpu.VMEM((B, W), jnp.float32),
            pltpu.VMEM((RPT,), jnp.float32),
            pltpu.SemaphoreType.DMA,
            pltpu.SemaphoreType.DMA,
            pltpu.SemaphoreType.DMA,
            pltpu.SemaphoreType.DMA,
            pltpu.SemaphoreType.DMA,
            pltpu.SemaphoreType.DMA,
        ],
    )(epack, m_i32, zeros)


def _gate_body(s_ref, m_ref, c_ref, dis_ref, bi, bf, bc, bo, wci, wcf, wco,
               h_out, c_out):
    dis = dis_ref[...]
    dis2 = dis * dis
    Cc = c_ref[...]
    Pi = dis * s_ref[0] + dis2 * m_ref[0]
    Pf = dis * s_ref[1] + dis2 * m_ref[1]
    Pc = dis * s_ref[2] + dis2 * m_ref[2]
    Po = dis * s_ref[3] + dis2 * m_ref[3]
    I = jax.nn.sigmoid(Pi + bi[...] + wci[...] * Cc)
    F = jax.nn.sigmoid(Pf + bf[...] + wcf[...] * Cc)
    T = jnp.tanh(Pc + bc[...])
    Cn = F * Cc + I * T
    O = jax.nn.sigmoid(Po + bo[...] + wco[...] * Cn)
    h_out[...] = O * jnp.tanh(Cn)
    c_out[...] = Cn


def _gating(s_ch, M4, C, dis_pad, bi, bf, bc, bo, wci, wcf, wco):
    vec = lambda: pl.BlockSpec((1, 128), lambda i: (0, 0))
    return pl.pallas_call(
        _gate_body,
        grid=(10,),
        in_specs=[
            pl.BlockSpec((4, 1000, 128), lambda i: (0, i, 0)),
            pl.BlockSpec((4, 1000, 128), lambda i: (0, i, 0)),
            pl.BlockSpec((1000, 128), lambda i: (i, 0)),
            pl.BlockSpec((1000, 1), lambda i: (i, 0)),
            vec(), vec(), vec(), vec(), vec(), vec(), vec(),
        ],
        out_specs=[
            pl.BlockSpec((1000, 128), lambda i: (i, 0)),
            pl.BlockSpec((1000, 128), lambda i: (i, 0)),
        ],
        out_shape=[
            jax.ShapeDtypeStruct((N, 128), jnp.float32),
            jax.ShapeDtypeStruct((N, 128), jnp.float32),
        ],
    )(s_ch, M4, C, dis_pad, bi, bf, bc, bo, wci, wcf, wco)


def kernel(X, edge_index, edge_weight, H, C,
           W_x_i, b_x_i, W_h_i, b_h_i,
           W_x_f, b_x_f, W_h_f, b_h_f,
           W_x_c, b_x_c, W_h_c, b_h_c,
           W_x_o, b_x_o, W_h_o, b_h_o,
           w_c_i, w_c_f, w_c_o, b_i, b_f, b_c, b_o):
    ei = edge_index.astype(jnp.int32)
    pad_i = jnp.zeros((EPAD - E,), jnp.int32)
    row_p = jnp.concatenate([ei[0], pad_i]).reshape(NS, NBLK, 1, B)
    col_p = jnp.concatenate([ei[1], pad_i]).reshape(NS, NBLK, 1, B)
    ew_p = jnp.concatenate(
        [lax.bitcast_convert_type(edge_weight, jnp.int32), pad_i]
    ).reshape(NS, NBLK, 1, B)
    epack = jnp.concatenate(
        [row_p, col_p, ew_p], axis=2).reshape(NS, NBLK, 3 * B)
    epack = jnp.concatenate(
        [epack, jnp.zeros((NS, 2, 3 * B), jnp.int32)], axis=1)

    Wx = jnp.concatenate([W_x_i, W_x_f, W_x_c, W_x_o], axis=1)
    Wh = jnp.concatenate([W_h_i, W_h_f, W_h_c, W_h_o], axis=1)
    M4 = _matmul(X, H, Wx, Wh)

    mperm = M4.astype(jnp.bfloat16).reshape(4, N, 4, 32)[..., _PERM32]
    m_i32 = lax.bitcast_convert_type(
        mperm.reshape(2 * NCH * N, WW, 2), jnp.int32)
    zeros = jnp.zeros((NPAD, W), jnp.float32)

    s_ch, dis_pad = _sc_call(epack, m_i32, zeros)
    dis2d = dis_pad.reshape(NPAD, 1)

    bi = (b_x_i + b_h_i).reshape(1, 128) + b_i
    bf = (b_x_f + b_h_f).reshape(1, 128) + b_f
    bc = (b_x_c + b_h_c).reshape(1, 128) + b_c
    bo = (b_x_o + b_h_o).reshape(1, 128) + b_o

    H_new, C_new = _gating(s_ch, M4, C, dis2d, bi, bf, bc, bo,
                           w_c_i, w_c_f, w_c_o)
    return (H_new, C_new)


# speedup vs baseline: 1.0021x; 1.0021x over previous
import jax
import jax.numpy as jnp
import numpy as np
from jax import lax
from jax.experimental import pallas as pl
from jax.experimental.pallas import tpu as pltpu
from jax.experimental.pallas import tpu_sc as plsc

N = 10000
E = 320000
DG = 512
W = 128
WW = W // 2
NCH = 2
NS = 16
B = 64
NBLK = 316
EPT = NBLK * B
EPAD = NS * EPT
NPAD = 10240
RPT = NPAD // NS

_PERM32 = np.stack([np.arange(16), np.arange(16) + 16], axis=1).reshape(32)


def _mm_body(x_ref, h_ref, wx_ref, wh_ref, o_ref):
    o_ref[0] = (
        jnp.dot(x_ref[...], wx_ref[...], preferred_element_type=jnp.float32)
        + jnp.dot(h_ref[...], wh_ref[...], preferred_element_type=jnp.float32)
    )


def _matmul(X, H, Wx, Wh):
    return pl.pallas_call(
        _mm_body,
        grid=(4, 10),
        in_specs=[
            pl.BlockSpec((1000, 128), lambda j, i: (i, 0)),
            pl.BlockSpec((1000, 128), lambda j, i: (i, 0)),
            pl.BlockSpec((128, 128), lambda j, i: (0, j)),
            pl.BlockSpec((128, 128), lambda j, i: (0, j)),
        ],
        out_specs=pl.BlockSpec((1, 1000, 128), lambda j, i: (j, i, 0)),
        out_shape=jax.ShapeDtypeStruct((4, N, 128), jnp.float32),
    )(X, H, Wx, Wh)


def _sc_body(ep_hbm, m_hbm, z_hbm,
             s_out, dis_out,
             ssp, degsp,
             ebuf0, ebuf1, idx0, idx1, ab0, ab1, cb0, cb1,
             dis_t, gbuf0, gbuf1, sbuf0, sbuf1, tmp_t,
             esem0, esem1, gsem0, gsem1, ssem0, ssem1):
    c = lax.axis_index("c")
    s = lax.axis_index("s")
    base = s * RPT

    ebufs = (ebuf0, ebuf1)
    idxs = (idx0, idx1)
    abs_ = (ab0, ab1)
    cbs = (cb0, cb1)
    gbufs = (gbuf0, gbuf1)
    sbufs = (sbuf0, sbuf1)
    esems = (esem0, esem1)
    gsems = (gsem0, gsem1)
    ssems = (ssem0, ssem1)
    NG = B // 16

    def estart(b, p):
        pltpu.async_copy(ep_hbm.at[s, b], ebufs[p], esems[p])

    def ewait(b, p):
        pltpu.make_async_copy(ep_hbm.at[s, b], ebufs[p], esems[p]).wait()

    def edrain():
        ewait(NBLK, 0)
        ewait(NBLK + 1, 1)

    def fill_ones(i, carry):
        tmp_t[pl.ds(i * 16, 16)] = jnp.ones((16,), jnp.float32)
        return carry
    lax.fori_loop(0, RPT // 16, fill_ones, 0)
    pltpu.sync_copy(tmp_t, degsp.at[pl.ds(base, RPT)])
    plsc.subcore_barrier()

    def deg_step(b, p, first):
        if not first:
            pltpu.make_async_copy(
                abs_[p], degsp.at[cbs[p]], ssems[p]).wait()
        ewait(b, p)

        def cvt(g, carry2):
            sl = pl.ds(g * 16, 16)
            abs_[p][sl] = lax.bitcast_convert_type(
                ebufs[p][pl.ds(2 * B + g * 16, 16)], jnp.float32)
            cbs[p][sl] = ebufs[p][pl.ds(B + g * 16, 16)]
            return carry2
        lax.fori_loop(0, NG, cvt, 0)
        estart(b + 2, p)
        pltpu.async_copy(abs_[p], degsp.at[cbs[p]], ssems[p], add=True)

    estart(0, 0)
    estart(1, 1)
    deg_step(0, 0, True)
    deg_step(1, 1, True)

    def deg_pipe(j, carry):
        deg_step(2 * j + 2, 0, False)
        deg_step(2 * j + 3, 1, False)
        return carry
    lax.fori_loop(0, NBLK // 2 - 1, deg_pipe, 0)
    for p in range(2):
        pltpu.make_async_copy(abs_[p], degsp.at[cbs[p]], ssems[p]).wait()
    edrain()
    plsc.subcore_barrier()

    pltpu.sync_copy(degsp.at[pl.ds(base, RPT)], tmp_t)

    def rsqrt_vec(i, carry):
        x = tmp_t[pl.ds(i * 16, 16)]
        ii = lax.bitcast_convert_type(x, jnp.int32)
        ii = jnp.int32(0x5F3759DF) - lax.shift_right_logical(ii, 1)
        y = lax.bitcast_convert_type(ii, jnp.float32)
        y = y * (1.5 - 0.5 * x * y * y)
        y = y * (1.5 - 0.5 * x * y * y)
        y = y * (1.5 - 0.5 * x * y * y)
        tmp_t[pl.ds(i * 16, 16)] = y
        return carry
    lax.fori_loop(0, RPT // 16, rsqrt_vec, 0)
    pltpu.sync_copy(tmp_t, degsp.at[pl.ds(base, RPT)])

    @pl.when(c == 0)
    def _():
        pltpu.sync_copy(tmp_t, dis_out.at[pl.ds(base, RPT)])
    plsc.subcore_barrier()

    pltpu.sync_copy(degsp, dis_t)

    def chunk(k, carry):
        ci = c * NCH + k

        pltpu.sync_copy(z_hbm.at[pl.ds(base, RPT)], ssp.at[pl.ds(base, RPT)])
        plsc.subcore_barrier()

        def issue(b, p, first):
            if not first:
                pltpu.make_async_copy(
                    sbufs[p], ssp.at[cbs[p]], ssems[p]).wait()
            ewait(b, p)

            def prep(g, carry3):
                sl = pl.ds(g * 16, 16)
                r16 = ebufs[p][sl]
                idxs[p][sl] = r16 + ci * N
                cbs[p][sl] = ebufs[p][pl.ds(B + g * 16, 16)]
                ew16 = lax.bitcast_convert_type(
                    ebufs[p][pl.ds(2 * B + g * 16, 16)], jnp.float32)
                abs_[p][sl] = ew16 * plsc.load_gather(dis_t, [r16])
                return carry3
            lax.fori_loop(0, NG, prep, 0)
            estart(b + 2, p)
            pltpu.async_copy(m_hbm.at[idxs[p]], gbufs[p], gsems[p])

        def retire(b, p):
            pltpu.make_async_copy(m_hbm.at[idxs[p]], gbufs[p], gsems[p]).wait()

            def grp(g, carry3):
                a16 = abs_[p][pl.ds(g * 16, 16)]
                for e in range(16):
                    ae = jnp.take_along_axis(
                        a16, jnp.full((16,), e, jnp.int32), axis=0)
                    r = g * 16 + e
                    for q in range(WW // 16):
                        wv = gbufs[p][r, pl.ds(q * 16, 16)]
                        lo = lax.bitcast_convert_type(
                            lax.shift_left(wv, 16), jnp.float32)
                        hi = lax.bitcast_convert_type(
                            wv & jnp.int32(-65536), jnp.float32)
                        sbufs[p][r, pl.ds(q * 32, 16)] = lo * ae
                        sbufs[p][r, pl.ds(q * 32 + 16, 16)] = hi * ae
                return carry3
            lax.fori_loop(0, NG, grp, 0)
            pltpu.async_copy(sbufs[p], ssp.at[cbs[p]], ssems[p], add=True)

        estart(0, 0)
        estart(1, 1)
        issue(0, 0, True)
        issue(1, 1, True)

        def pipe(j, carry2):
            retire(2 * j, 0)
            issue(2 * j + 2, 0, False)
            retire(2 * j + 1, 1)
            issue(2 * j + 3, 1, False)
            return carry2
        lax.fori_loop(0, NBLK // 2 - 1, pipe, 0)
        retire(NBLK - 2, 0)
        retire(NBLK - 1, 1)
        for p in range(2):
            pltpu.make_async_copy(
                sbufs[p], ssp.at[cbs[p]], ssems[p]).wait()
        edrain()

        plsc.subcore_barrier()
        pltpu.sync_copy(ssp.at[pl.ds(base, RPT)],
                        s_out.at[ci, pl.ds(base, RPT)])
        plsc.subcore_barrier()
        return carry
    lax.fori_loop(0, NCH, chunk, 0)


def _sc_call(epack, m_i32, zeros):
    mesh = plsc.VectorSubcoreMesh(core_axis_name="c", subcore_axis_name="s",
                                  num_cores=2, num_subcores=NS)
    return pl.kernel(
        _sc_body,
        out_type=(
            jax.ShapeDtypeStruct((2 * NCH, NPAD, W), jnp.float32),
            jax.ShapeDtypeStruct((NPAD,), jnp.float32),
        ),
        mesh=mesh,
        compiler_params=pltpu.CompilerParams(needs_layout_passes=False,
                                             use_tc_tiling_on_sc=False),
        scratch_types=[
            pltpu.VMEM_SHARED((NPAD, W), jnp.float32),
            pltpu.VMEM_SHARED((NPAD,), jnp.float32),
            pltpu.VMEM((3 * B,), jnp.int32),
            pltpu.VMEM((3 * B,), jnp.int32),
            pltpu.VMEM((B,), jnp.int32),
            pltpu.VMEM((B,), jnp.int32),
            pltpu.VMEM((B,), jnp.float32),
            pltpu.VMEM((B,), jnp.float32),
            pltpu.VMEM((B,), jnp.int32),
            pltpu.VMEM((B,), jnp.int32),
            pltpu.VMEM((NPAD,), jnp.float32),
            pltpu.VMEM((B, WW), jnp.int32),
            pltpu.VMEM((B, WW), jnp.int32),
            pltpu.VMEM((B, W), jnp.float32),
            pltpu.VMEM((B, W), jnp.float32),
            pltpu.VMEM((RPT,), jnp.float32),
            pltpu.SemaphoreType.DMA,
            pltpu.SemaphoreType.DMA,
            pltpu.SemaphoreType.DMA,
            pltpu.SemaphoreType.DMA,
            pltpu.SemaphoreType.DMA,
            pltpu.SemaphoreType.DMA,
        ],
    )(epack, m_i32, zeros)


def _gate_body(s_ref, m_ref, c_ref, dis_ref, bi, bf, bc, bo, wci, wcf, wco,
               h_out, c_out):
    dis = dis_ref[...]
    dis2 = dis * dis
    Cc = c_ref[...]
    Pi = dis * s_ref[0] + dis2 * m_ref[0]
    Pf = dis * s_ref[1] + dis2 * m_ref[1]
    Pc = dis * s_ref[2] + dis2 * m_ref[2]
    Po = dis * s_ref[3] + dis2 * m_ref[3]
    I = jax.nn.sigmoid(Pi + bi[...] + wci[...] * Cc)
    F = jax.nn.sigmoid(Pf + bf[...] + wcf[...] * Cc)
    T = jnp.tanh(Pc + bc[...])
    Cn = F * Cc + I * T
    O = jax.nn.sigmoid(Po + bo[...] + wco[...] * Cn)
    h_out[...] = O * jnp.tanh(Cn)
    c_out[...] = Cn


def _gating(s_ch, M4, C, dis_pad, bi, bf, bc, bo, wci, wcf, wco):
    vec = lambda: pl.BlockSpec((1, 128), lambda i: (0, 0))
    return pl.pallas_call(
        _gate_body,
        grid=(10,),
        in_specs=[
            pl.BlockSpec((4, 1000, 128), lambda i: (0, i, 0)),
            pl.BlockSpec((4, 1000, 128), lambda i: (0, i, 0)),
            pl.BlockSpec((1000, 128), lambda i: (i, 0)),
            pl.BlockSpec((1000, 1), lambda i: (i, 0)),
            vec(), vec(), vec(), vec(), vec(), vec(), vec(),
        ],
        out_specs=[
            pl.BlockSpec((1000, 128), lambda i: (i, 0)),
            pl.BlockSpec((1000, 128), lambda i: (i, 0)),
        ],
        out_shape=[
            jax.ShapeDtypeStruct((N, 128), jnp.float32),
            jax.ShapeDtypeStruct((N, 128), jnp.float32),
        ],
    )(s_ch, M4, C, dis_pad, bi, bf, bc, bo, wci, wcf, wco)


def kernel(X, edge_index, edge_weight, H, C,
           W_x_i, b_x_i, W_h_i, b_h_i,
           W_x_f, b_x_f, W_h_f, b_h_f,
           W_x_c, b_x_c, W_h_c, b_h_c,
           W_x_o, b_x_o, W_h_o, b_h_o,
           w_c_i, w_c_f, w_c_o, b_i, b_f, b_c, b_o):
    ei = edge_index.astype(jnp.int32)
    pad_i = jnp.zeros((EPAD - E,), jnp.int32)
    row_p = jnp.concatenate([ei[0], pad_i]).reshape(NS, NBLK, 1, B)
    col_p = jnp.concatenate([ei[1], pad_i]).reshape(NS, NBLK, 1, B)
    ew_p = jnp.concatenate(
        [lax.bitcast_convert_type(edge_weight, jnp.int32), pad_i]
    ).reshape(NS, NBLK, 1, B)
    epack = jnp.concatenate(
        [row_p, col_p, ew_p], axis=2).reshape(NS, NBLK, 3 * B)
    epack = jnp.concatenate(
        [epack, jnp.zeros((NS, 2, 3 * B), jnp.int32)], axis=1)

    Wx = jnp.concatenate([W_x_i, W_x_f, W_x_c, W_x_o], axis=1)
    Wh = jnp.concatenate([W_h_i, W_h_f, W_h_c, W_h_o], axis=1)
    M4 = _matmul(X, H, Wx, Wh)

    mperm = M4.astype(jnp.bfloat16).reshape(4, N, 4, 32)[..., _PERM32]
    m_i32 = lax.bitcast_convert_type(
        mperm.reshape(2 * NCH * N, WW, 2), jnp.int32)
    zeros = jnp.zeros((NPAD, W), jnp.float32)

    s_ch, dis_pad = _sc_call(epack, m_i32, zeros)
    dis2d = dis_pad.reshape(NPAD, 1)

    bi = (b_x_i + b_h_i).reshape(1, 128) + b_i
    bf = (b_x_f + b_h_f).reshape(1, 128) + b_f
    bc = (b_x_c + b_h_c).reshape(1, 128) + b_c
    bo = (b_x_o + b_h_o).reshape(1, 128) + b_o

    H_new, C_new = _gating(s_ch, M4, C, dis2d, bi, bf, bc, bo,
                           w_c_i, w_c_f, w_c_o)
    return (H_new, C_new)


# speedup vs baseline: 1.8808x; 1.8769x over previous
import jax
import jax.numpy as jnp
from jax import lax
from jax.experimental import pallas as pl
from jax.experimental.pallas import tpu as pltpu
from jax.experimental.pallas import tpu_sc as plsc

N = 10000
E = 320000
DG = 512
W = 128
NCH = 2
NS = 16
B = 128
NBLK = 158
EPT = NBLK * B
EPAD = NS * EPT
NPAD = 10240
RPT = NPAD // NS


def _mm_body(x_ref, h_ref, wx_ref, wh_ref, o_ref):
    o_ref[0] = (
        jnp.dot(x_ref[...], wx_ref[...], preferred_element_type=jnp.float32)
        + jnp.dot(h_ref[...], wh_ref[...], preferred_element_type=jnp.float32)
    )


def _matmul(X, H, Wx, Wh):
    return pl.pallas_call(
        _mm_body,
        grid=(4, 10),
        in_specs=[
            pl.BlockSpec((1000, 128), lambda j, i: (i, 0)),
            pl.BlockSpec((1000, 128), lambda j, i: (i, 0)),
            pl.BlockSpec((128, 128), lambda j, i: (0, j)),
            pl.BlockSpec((128, 128), lambda j, i: (0, j)),
        ],
        out_specs=pl.BlockSpec((1, 1000, 128), lambda j, i: (j, i, 0)),
        out_shape=jax.ShapeDtypeStruct((4, N, 128), jnp.float32),
    )(X, H, Wx, Wh)


def _sc_body(ep_hbm, m_hbm, z_hbm,
             s_out, dis_out,
             ssp, degsp,
             ebuf0, ebuf1, idx0, idx1, ab0, ab1, cb0, cb1,
             dis_t, gbuf0, gbuf1, tmp_t,
             esem0, esem1, gsem0, gsem1, ssem0, ssem1):
    c = lax.axis_index("c")
    s = lax.axis_index("s")
    base = s * RPT

    ebufs = (ebuf0, ebuf1)
    idxs = (idx0, idx1)
    abs_ = (ab0, ab1)
    cbs = (cb0, cb1)
    gbufs = (gbuf0, gbuf1)
    esems = (esem0, esem1)
    gsems = (gsem0, gsem1)
    ssems = (ssem0, ssem1)

    def estart(b, p):
        pltpu.async_copy(ep_hbm.at[s, b], ebufs[p], esems[p])

    def ewait(b, p):
        pltpu.make_async_copy(ep_hbm.at[s, b], ebufs[p], esems[p]).wait()

    def edrain():
        ewait(NBLK, 0)
        ewait(NBLK + 1, 1)

    def fill_ones(i, carry):
        tmp_t[pl.ds(i * 16, 16)] = jnp.ones((16,), jnp.float32)
        return carry
    lax.fori_loop(0, RPT // 16, fill_ones, 0)
    pltpu.sync_copy(tmp_t, degsp.at[pl.ds(base, RPT)])
    plsc.subcore_barrier()

    def deg_step(b, p, first):
        if not first:
            pltpu.make_async_copy(
                abs_[p], degsp.at[cbs[p]], ssems[p]).wait()
        ewait(b, p)

        def cvt(g, carry2):
            sl = pl.ds(g * 16, 16)
            abs_[p][sl] = lax.bitcast_convert_type(
                ebufs[p][2, sl], jnp.float32)
            cbs[p][sl] = ebufs[p][1, sl]
            return carry2
        lax.fori_loop(0, B // 16, cvt, 0)
        estart(b + 2, p)
        pltpu.async_copy(abs_[p], degsp.at[cbs[p]], ssems[p], add=True)

    estart(0, 0)
    estart(1, 1)
    deg_step(0, 0, True)
    deg_step(1, 1, True)

    def deg_pipe(j, carry):
        deg_step(2 * j + 2, 0, False)
        deg_step(2 * j + 3, 1, False)
        return carry
    lax.fori_loop(0, NBLK // 2 - 1, deg_pipe, 0)
    for p in range(2):
        pltpu.make_async_copy(abs_[p], degsp.at[cbs[p]], ssems[p]).wait()
    edrain()
    plsc.subcore_barrier()

    pltpu.sync_copy(degsp.at[pl.ds(base, RPT)], tmp_t)

    def rsqrt_vec(i, carry):
        x = tmp_t[pl.ds(i * 16, 16)]
        ii = lax.bitcast_convert_type(x, jnp.int32)
        ii = jnp.int32(0x5F3759DF) - lax.shift_right_logical(ii, 1)
        y = lax.bitcast_convert_type(ii, jnp.float32)
        y = y * (1.5 - 0.5 * x * y * y)
        y = y * (1.5 - 0.5 * x * y * y)
        y = y * (1.5 - 0.5 * x * y * y)
        tmp_t[pl.ds(i * 16, 16)] = y
        return carry
    lax.fori_loop(0, RPT // 16, rsqrt_vec, 0)
    pltpu.sync_copy(tmp_t, degsp.at[pl.ds(base, RPT)])

    @pl.when(c == 0)
    def _():
        pltpu.sync_copy(tmp_t, dis_out.at[pl.ds(base, RPT)])
    plsc.subcore_barrier()

    pltpu.sync_copy(degsp, dis_t)

    def chunk(k, carry):
        ci = c * NCH + k

        pltpu.sync_copy(z_hbm.at[pl.ds(base, RPT)], ssp.at[pl.ds(base, RPT)])
        plsc.subcore_barrier()

        def issue(b, p, first):
            if not first:
                for h in range(2):
                    pltpu.make_async_copy(
                        gbufs[p].at[pl.ds(h * (B // 2), B // 2)],
                        ssp.at[cbs[p].at[pl.ds(h * (B // 2), B // 2)]],
                        ssems[p]).wait()
            ewait(b, p)

            def prep(g, carry3):
                sl = pl.ds(g * 16, 16)
                r16 = ebufs[p][0, sl]
                idxs[p][sl] = r16 + ci * N
                cbs[p][sl] = ebufs[p][1, sl]
                ew16 = lax.bitcast_convert_type(ebufs[p][2, sl], jnp.float32)
                abs_[p][sl] = ew16 * plsc.load_gather(dis_t, [r16])
                return carry3
            lax.fori_loop(0, B // 16, prep, 0)
            estart(b + 2, p)
            pltpu.async_copy(m_hbm.at[idxs[p]], gbufs[p], gsems[p])

        def retire(b, p):
            pltpu.make_async_copy(m_hbm.at[idxs[p]], gbufs[p], gsems[p]).wait()

            def grp(g, carry3):
                a16 = abs_[p][pl.ds(g * 16, 16)]
                for e in range(16):
                    ae = jnp.take_along_axis(
                        a16, jnp.full((16,), e, jnp.int32), axis=0)
                    for q in range(W // 16):
                        v = gbufs[p][g * 16 + e, pl.ds(q * 16, 16)]
                        gbufs[p][g * 16 + e, pl.ds(q * 16, 16)] = v * ae
                return carry3
            lax.fori_loop(0, B // 32, grp, 0)
            pltpu.async_copy(gbufs[p].at[pl.ds(0, B // 2)],
                             ssp.at[cbs[p].at[pl.ds(0, B // 2)]],
                             ssems[p], add=True)
            lax.fori_loop(B // 32, B // 16, grp, 0)
            pltpu.async_copy(gbufs[p].at[pl.ds(B // 2, B // 2)],
                             ssp.at[cbs[p].at[pl.ds(B // 2, B // 2)]],
                             ssems[p], add=True)

        estart(0, 0)
        estart(1, 1)
        issue(0, 0, True)
        issue(1, 1, True)

        def pipe(j, carry2):
            retire(2 * j, 0)
            issue(2 * j + 2, 0, False)
            retire(2 * j + 1, 1)
            issue(2 * j + 3, 1, False)
            return carry2
        lax.fori_loop(0, NBLK // 2 - 1, pipe, 0)
        retire(NBLK - 2, 0)
        retire(NBLK - 1, 1)
        for p in range(2):
            for h in range(2):
                pltpu.make_async_copy(
                    gbufs[p].at[pl.ds(h * (B // 2), B // 2)],
                    ssp.at[cbs[p].at[pl.ds(h * (B // 2), B // 2)]],
                    ssems[p]).wait()
        edrain()

        plsc.subcore_barrier()
        pltpu.sync_copy(ssp.at[pl.ds(base, RPT)],
                        s_out.at[ci, pl.ds(base, RPT)])
        plsc.subcore_barrier()
        return carry
    lax.fori_loop(0, NCH, chunk, 0)


def _sc_call(epack, m_flat, zeros):
    mesh = plsc.VectorSubcoreMesh(core_axis_name="c", subcore_axis_name="s",
                                  num_cores=2, num_subcores=NS)
    return pl.kernel(
        _sc_body,
        out_type=(
            jax.ShapeDtypeStruct((2 * NCH, NPAD, W), jnp.float32),
            jax.ShapeDtypeStruct((NPAD,), jnp.float32),
        ),
        mesh=mesh,
        compiler_params=pltpu.CompilerParams(needs_layout_passes=False, use_tc_tiling_on_sc=False),
        scratch_types=[
            pltpu.VMEM_SHARED((NPAD, W), jnp.float32),
            pltpu.VMEM_SHARED((NPAD,), jnp.float32),
            pltpu.VMEM((3, B), jnp.int32),
            pltpu.VMEM((3, B), jnp.int32),
            pltpu.VMEM((B,), jnp.int32),
            pltpu.VMEM((B,), jnp.int32),
            pltpu.VMEM((B,), jnp.float32),
            pltpu.VMEM((B,), jnp.float32),
            pltpu.VMEM((B,), jnp.int32),
            pltpu.VMEM((B,), jnp.int32),
            pltpu.VMEM((NPAD,), jnp.float32),
            pltpu.VMEM((B, W), jnp.float32),
            pltpu.VMEM((B, W), jnp.float32),
            pltpu.VMEM((RPT,), jnp.float32),
            pltpu.SemaphoreType.DMA,
            pltpu.SemaphoreType.DMA,
            pltpu.SemaphoreType.DMA,
            pltpu.SemaphoreType.DMA,
            pltpu.SemaphoreType.DMA,
            pltpu.SemaphoreType.DMA,
        ],
    )(epack, m_flat, zeros)


def _gate_body(s_ref, m_ref, c_ref, dis_ref, bi, bf, bc, bo, wci, wcf, wco,
               h_out, c_out):
    dis = dis_ref[...]
    dis2 = dis * dis
    Cc = c_ref[...]
    Pi = dis * s_ref[0] + dis2 * m_ref[0]
    Pf = dis * s_ref[1] + dis2 * m_ref[1]
    Pc = dis * s_ref[2] + dis2 * m_ref[2]
    Po = dis * s_ref[3] + dis2 * m_ref[3]
    I = jax.nn.sigmoid(Pi + bi[...] + wci[...] * Cc)
    F = jax.nn.sigmoid(Pf + bf[...] + wcf[...] * Cc)
    T = jnp.tanh(Pc + bc[...])
    Cn = F * Cc + I * T
    O = jax.nn.sigmoid(Po + bo[...] + wco[...] * Cn)
    h_out[...] = O * jnp.tanh(Cn)
    c_out[...] = Cn


def _gating(s_ch, M4, C, dis_pad, bi, bf, bc, bo, wci, wcf, wco):
    vec = lambda: pl.BlockSpec((1, 128), lambda i: (0, 0))
    return pl.pallas_call(
        _gate_body,
        grid=(10,),
        in_specs=[
            pl.BlockSpec((4, 1000, 128), lambda i: (0, i, 0)),
            pl.BlockSpec((4, 1000, 128), lambda i: (0, i, 0)),
            pl.BlockSpec((1000, 128), lambda i: (i, 0)),
            pl.BlockSpec((1000, 1), lambda i: (i, 0)),
            vec(), vec(), vec(), vec(), vec(), vec(), vec(),
        ],
        out_specs=[
            pl.BlockSpec((1000, 128), lambda i: (i, 0)),
            pl.BlockSpec((1000, 128), lambda i: (i, 0)),
        ],
        out_shape=[
            jax.ShapeDtypeStruct((N, 128), jnp.float32),
            jax.ShapeDtypeStruct((N, 128), jnp.float32),
        ],
    )(s_ch, M4, C, dis_pad, bi, bf, bc, bo, wci, wcf, wco)


def kernel(X, edge_index, edge_weight, H, C,
           W_x_i, b_x_i, W_h_i, b_h_i,
           W_x_f, b_x_f, W_h_f, b_h_f,
           W_x_c, b_x_c, W_h_c, b_h_c,
           W_x_o, b_x_o, W_h_o, b_h_o,
           w_c_i, w_c_f, w_c_o, b_i, b_f, b_c, b_o):
    ei = edge_index.astype(jnp.int32)
    pad_i = jnp.zeros((EPAD - E,), jnp.int32)
    row_p = jnp.concatenate([ei[0], pad_i]).reshape(NS, NBLK, 1, B)
    col_p = jnp.concatenate([ei[1], pad_i]).reshape(NS, NBLK, 1, B)
    ew_p = jnp.concatenate(
        [lax.bitcast_convert_type(edge_weight, jnp.int32), pad_i]
    ).reshape(NS, NBLK, 1, B)
    epack = jnp.concatenate([row_p, col_p, ew_p], axis=2)
    epack = jnp.concatenate(
        [epack, jnp.zeros((NS, 2, 3, B), jnp.int32)], axis=1)

    Wx = jnp.concatenate([W_x_i, W_x_f, W_x_c, W_x_o], axis=1)
    Wh = jnp.concatenate([W_h_i, W_h_f, W_h_c, W_h_o], axis=1)
    M4 = _matmul(X, H, Wx, Wh)

    m_flat = M4.reshape(2 * NCH * N, W)
    zeros = jnp.zeros((NPAD, W), jnp.float32)

    s_ch, dis_pad = _sc_call(epack, m_flat, zeros)
    dis2d = dis_pad.reshape(NPAD, 1)

    bi = (b_x_i + b_h_i).reshape(1, 128) + b_i
    bf = (b_x_f + b_h_f).reshape(1, 128) + b_f
    bc = (b_x_c + b_h_c).reshape(1, 128) + b_c
    bo = (b_x_o + b_h_o).reshape(1, 128) + b_o

    H_new, C_new = _gating(s_ch, M4, C, dis2d, bi, bf, bc, bo,
                           w_c_i, w_c_f, w_c_o)
    return (H_new, C_new)


# speedup vs baseline: 1.9312x; 1.0268x over previous
import jax
import jax.numpy as jnp
from jax import lax
from jax.experimental import pallas as pl
from jax.experimental.pallas import tpu as pltpu
from jax.experimental.pallas import tpu_sc as plsc

N = 10000
E = 320000
DG = 512
W = 128
NCH = 2
NS = 16
B = 128
NBLK = 158
EPT = NBLK * B
EPAD = NS * EPT
NPAD = 10240
RPT = NPAD // NS


def _mm_body(x_ref, h_ref, wx_ref, wh_ref, o_ref):
    o_ref[0] = (
        jnp.dot(x_ref[...], wx_ref[...], preferred_element_type=jnp.float32)
        + jnp.dot(h_ref[...], wh_ref[...], preferred_element_type=jnp.float32)
    )


def _matmul(X, H, Wx, Wh):
    return pl.pallas_call(
        _mm_body,
        grid=(4, 10),
        in_specs=[
            pl.BlockSpec((1000, 128), lambda j, i: (i, 0)),
            pl.BlockSpec((1000, 128), lambda j, i: (i, 0)),
            pl.BlockSpec((128, 128), lambda j, i: (0, j)),
            pl.BlockSpec((128, 128), lambda j, i: (0, j)),
        ],
        out_specs=pl.BlockSpec((1, 1000, 128), lambda j, i: (j, i, 0)),
        out_shape=jax.ShapeDtypeStruct((4, N, 128), jnp.float32),
    )(X, H, Wx, Wh)


def _sc_body(ep_hbm, m_hbm, z_hbm,
             s_out, dis_out,
             ssp, degsp,
             ebuf0, ebuf1, idx0, idx1, ab0, ab1, cb0, cb1,
             dis_t, gbuf0, gbuf1, tmp_t,
             esem0, esem1, gsem0, gsem1, ssem0, ssem1):
    c = lax.axis_index("c")
    s = lax.axis_index("s")
    base = s * RPT

    ebufs = (ebuf0, ebuf1)
    idxs = (idx0, idx1)
    abs_ = (ab0, ab1)
    cbs = (cb0, cb1)
    gbufs = (gbuf0, gbuf1)
    esems = (esem0, esem1)
    gsems = (gsem0, gsem1)
    ssems = (ssem0, ssem1)

    def estart(b, p):
        pltpu.async_copy(ep_hbm.at[s, b], ebufs[p], esems[p])

    def ewait(b, p):
        pltpu.make_async_copy(ep_hbm.at[s, b], ebufs[p], esems[p]).wait()

    def edrain():
        ewait(NBLK, 0)
        ewait(NBLK + 1, 1)

    def fill_ones(i, carry):
        tmp_t[pl.ds(i * 16, 16)] = jnp.ones((16,), jnp.float32)
        return carry
    lax.fori_loop(0, RPT // 16, fill_ones, 0)
    pltpu.sync_copy(tmp_t, degsp.at[pl.ds(base, RPT)])
    plsc.subcore_barrier()

    def deg_step(b, p, first):
        if not first:
            pltpu.make_async_copy(
                abs_[p], degsp.at[cbs[p]], ssems[p]).wait()
        ewait(b, p)

        def cvt(g, carry2):
            sl = pl.ds(g * 16, 16)
            abs_[p][sl] = lax.bitcast_convert_type(
                ebufs[p][2, sl], jnp.float32)
            cbs[p][sl] = ebufs[p][1, sl]
            return carry2
        lax.fori_loop(0, B // 16, cvt, 0)
        estart(b + 2, p)
        pltpu.async_copy(abs_[p], degsp.at[cbs[p]], ssems[p], add=True)

    estart(0, 0)
    estart(1, 1)
    deg_step(0, 0, True)
    deg_step(1, 1, True)

    def deg_pipe(j, carry):
        deg_step(2 * j + 2, 0, False)
        deg_step(2 * j + 3, 1, False)
        return carry
    lax.fori_loop(0, NBLK // 2 - 1, deg_pipe, 0)
    for p in range(2):
        pltpu.make_async_copy(abs_[p], degsp.at[cbs[p]], ssems[p]).wait()
    edrain()
    plsc.subcore_barrier()

    pltpu.sync_copy(degsp.at[pl.ds(base, RPT)], tmp_t)

    def rsqrt_vec(i, carry):
        x = tmp_t[pl.ds(i * 16, 16)]
        ii = lax.bitcast_convert_type(x, jnp.int32)
        ii = jnp.int32(0x5F3759DF) - lax.shift_right_logical(ii, 1)
        y = lax.bitcast_convert_type(ii, jnp.float32)
        y = y * (1.5 - 0.5 * x * y * y)
        y = y * (1.5 - 0.5 * x * y * y)
        y = y * (1.5 - 0.5 * x * y * y)
        tmp_t[pl.ds(i * 16, 16)] = y
        return carry
    lax.fori_loop(0, RPT // 16, rsqrt_vec, 0)
    pltpu.sync_copy(tmp_t, degsp.at[pl.ds(base, RPT)])

    @pl.when(c == 0)
    def _():
        pltpu.sync_copy(tmp_t, dis_out.at[pl.ds(base, RPT)])
    plsc.subcore_barrier()

    pltpu.sync_copy(degsp, dis_t)

    def chunk(k, carry):
        ci = c * NCH + k

        pltpu.sync_copy(z_hbm.at[pl.ds(base, RPT)], ssp.at[pl.ds(base, RPT)])
        plsc.subcore_barrier()

        def issue(b, p, first):
            if not first:
                for h in range(2):
                    pltpu.make_async_copy(
                        gbufs[p].at[pl.ds(h * (B // 2), B // 2)],
                        ssp.at[cbs[p].at[pl.ds(h * (B // 2), B // 2)]],
                        ssems[p]).wait()
            ewait(b, p)

            def prep(g, carry3):
                sl = pl.ds(g * 16, 16)
                r16 = ebufs[p][0, sl]
                idxs[p][sl] = r16 + ci * N
                cbs[p][sl] = ebufs[p][1, sl]
                ew16 = lax.bitcast_convert_type(ebufs[p][2, sl], jnp.float32)
                abs_[p][sl] = ew16 * plsc.load_gather(dis_t, [r16])
                return carry3
            lax.fori_loop(0, B // 16, prep, 0)
            estart(b + 2, p)
            for h in range(2):
                pltpu.async_copy(
                    m_hbm.at[idxs[p].at[pl.ds(h * (B // 2), B // 2)]],
                    gbufs[p].at[pl.ds(h * (B // 2), B // 2)], gsems[p])

        def retire(b, p):
            def gwait(h):
                pltpu.make_async_copy(
                    m_hbm.at[idxs[p].at[pl.ds(h * (B // 2), B // 2)]],
                    gbufs[p].at[pl.ds(h * (B // 2), B // 2)], gsems[p]).wait()

            def grp(g, carry3):
                a16 = abs_[p][pl.ds(g * 16, 16)]
                for e in range(16):
                    ae = jnp.take_along_axis(
                        a16, jnp.full((16,), e, jnp.int32), axis=0)
                    for q in range(W // 16):
                        v = gbufs[p][g * 16 + e, pl.ds(q * 16, 16)]
                        gbufs[p][g * 16 + e, pl.ds(q * 16, 16)] = v * ae
                return carry3
            gwait(0)
            lax.fori_loop(0, B // 32, grp, 0)
            pltpu.async_copy(gbufs[p].at[pl.ds(0, B // 2)],
                             ssp.at[cbs[p].at[pl.ds(0, B // 2)]],
                             ssems[p], add=True)
            gwait(1)
            lax.fori_loop(B // 32, B // 16, grp, 0)
            pltpu.async_copy(gbufs[p].at[pl.ds(B // 2, B // 2)],
                             ssp.at[cbs[p].at[pl.ds(B // 2, B // 2)]],
                             ssems[p], add=True)

        estart(0, 0)
        estart(1, 1)
        issue(0, 0, True)
        issue(1, 1, True)

        def pipe(j, carry2):
            retire(2 * j, 0)
            issue(2 * j + 2, 0, False)
            retire(2 * j + 1, 1)
            issue(2 * j + 3, 1, False)
            return carry2
        lax.fori_loop(0, NBLK // 2 - 1, pipe, 0)
        retire(NBLK - 2, 0)
        retire(NBLK - 1, 1)
        for p in range(2):
            for h in range(2):
                pltpu.make_async_copy(
                    gbufs[p].at[pl.ds(h * (B // 2), B // 2)],
                    ssp.at[cbs[p].at[pl.ds(h * (B // 2), B // 2)]],
                    ssems[p]).wait()
        edrain()

        plsc.subcore_barrier()
        pltpu.sync_copy(ssp.at[pl.ds(base, RPT)],
                        s_out.at[ci, pl.ds(base, RPT)])
        plsc.subcore_barrier()
        return carry
    lax.fori_loop(0, NCH, chunk, 0)


def _sc_call(epack, m_flat, zeros):
    mesh = plsc.VectorSubcoreMesh(core_axis_name="c", subcore_axis_name="s",
                                  num_cores=2, num_subcores=NS)
    return pl.kernel(
        _sc_body,
        out_type=(
            jax.ShapeDtypeStruct((2 * NCH, NPAD, W), jnp.float32),
            jax.ShapeDtypeStruct((NPAD,), jnp.float32),
        ),
        mesh=mesh,
        compiler_params=pltpu.CompilerParams(needs_layout_passes=False, use_tc_tiling_on_sc=False),
        scratch_types=[
            pltpu.VMEM_SHARED((NPAD, W), jnp.float32),
            pltpu.VMEM_SHARED((NPAD,), jnp.float32),
            pltpu.VMEM((3, B), jnp.int32),
            pltpu.VMEM((3, B), jnp.int32),
            pltpu.VMEM((B,), jnp.int32),
            pltpu.VMEM((B,), jnp.int32),
            pltpu.VMEM((B,), jnp.float32),
            pltpu.VMEM((B,), jnp.float32),
            pltpu.VMEM((B,), jnp.int32),
            pltpu.VMEM((B,), jnp.int32),
            pltpu.VMEM((NPAD,), jnp.float32),
            pltpu.VMEM((B, W), jnp.float32),
            pltpu.VMEM((B, W), jnp.float32),
            pltpu.VMEM((RPT,), jnp.float32),
            pltpu.SemaphoreType.DMA,
            pltpu.SemaphoreType.DMA,
            pltpu.SemaphoreType.DMA,
            pltpu.SemaphoreType.DMA,
            pltpu.SemaphoreType.DMA,
            pltpu.SemaphoreType.DMA,
        ],
    )(epack, m_flat, zeros)


def _gate_body(s_ref, m_ref, c_ref, dis_ref, bi, bf, bc, bo, wci, wcf, wco,
               h_out, c_out):
    dis = dis_ref[...]
    dis2 = dis * dis
    Cc = c_ref[...]
    Pi = dis * s_ref[0] + dis2 * m_ref[0]
    Pf = dis * s_ref[1] + dis2 * m_ref[1]
    Pc = dis * s_ref[2] + dis2 * m_ref[2]
    Po = dis * s_ref[3] + dis2 * m_ref[3]
    I = jax.nn.sigmoid(Pi + bi[...] + wci[...] * Cc)
    F = jax.nn.sigmoid(Pf + bf[...] + wcf[...] * Cc)
    T = jnp.tanh(Pc + bc[...])
    Cn = F * Cc + I * T
    O = jax.nn.sigmoid(Po + bo[...] + wco[...] * Cn)
    h_out[...] = O * jnp.tanh(Cn)
    c_out[...] = Cn


def _gating(s_ch, M4, C, dis_pad, bi, bf, bc, bo, wci, wcf, wco):
    vec = lambda: pl.BlockSpec((1, 128), lambda i: (0, 0))
    return pl.pallas_call(
        _gate_body,
        grid=(10,),
        in_specs=[
            pl.BlockSpec((4, 1000, 128), lambda i: (0, i, 0)),
            pl.BlockSpec((4, 1000, 128), lambda i: (0, i, 0)),
            pl.BlockSpec((1000, 128), lambda i: (i, 0)),
            pl.BlockSpec((1000, 1), lambda i: (i, 0)),
            vec(), vec(), vec(), vec(), vec(), vec(), vec(),
        ],
        out_specs=[
            pl.BlockSpec((1000, 128), lambda i: (i, 0)),
            pl.BlockSpec((1000, 128), lambda i: (i, 0)),
        ],
        out_shape=[
            jax.ShapeDtypeStruct((N, 128), jnp.float32),
            jax.ShapeDtypeStruct((N, 128), jnp.float32),
        ],
    )(s_ch, M4, C, dis_pad, bi, bf, bc, bo, wci, wcf, wco)


def kernel(X, edge_index, edge_weight, H, C,
           W_x_i, b_x_i, W_h_i, b_h_i,
           W_x_f, b_x_f, W_h_f, b_h_f,
           W_x_c, b_x_c, W_h_c, b_h_c,
           W_x_o, b_x_o, W_h_o, b_h_o,
           w_c_i, w_c_f, w_c_o, b_i, b_f, b_c, b_o):
    ei = edge_index.astype(jnp.int32)
    pad_i = jnp.zeros((EPAD - E,), jnp.int32)
    row_p = jnp.concatenate([ei[0], pad_i]).reshape(NS, NBLK, 1, B)
    col_p = jnp.concatenate([ei[1], pad_i]).reshape(NS, NBLK, 1, B)
    ew_p = jnp.concatenate(
        [lax.bitcast_convert_type(edge_weight, jnp.int32), pad_i]
    ).reshape(NS, NBLK, 1, B)
    epack = jnp.concatenate([row_p, col_p, ew_p], axis=2)
    epack = jnp.concatenate(
        [epack, jnp.zeros((NS, 2, 3, B), jnp.int32)], axis=1)

    Wx = jnp.concatenate([W_x_i, W_x_f, W_x_c, W_x_o], axis=1)
    Wh = jnp.concatenate([W_h_i, W_h_f, W_h_c, W_h_o], axis=1)
    M4 = _matmul(X, H, Wx, Wh)

    m_flat = M4.reshape(2 * NCH * N, W)
    zeros = jnp.zeros((NPAD, W), jnp.float32)

    s_ch, dis_pad = _sc_call(epack, m_flat, zeros)
    dis2d = dis_pad.reshape(NPAD, 1)

    bi = (b_x_i + b_h_i).reshape(1, 128) + b_i
    bf = (b_x_f + b_h_f).reshape(1, 128) + b_f
    bc = (b_x_c + b_h_c).reshape(1, 128) + b_c
    bo = (b_x_o + b_h_o).reshape(1, 128) + b_o

    H_new, C_new = _gating(s_ch, M4, C, dis2d, bi, bf, bc, bo,
                           w_c_i, w_c_f, w_c_o)
    return (H_new, C_new)


# speedup vs baseline: 1.9891x; 1.0299x over previous
import jax
import jax.numpy as jnp
from jax import lax
from jax.experimental import pallas as pl
from jax.experimental.pallas import tpu as pltpu
from jax.experimental.pallas import tpu_sc as plsc

N = 10000
E = 320000
DG = 512
W = 128
NCH = 2
NS = 16
B = 128
NBLK = 158
EPT = NBLK * B
EPAD = NS * EPT
NPAD = 10240
RPT = NPAD // NS


def _mm_body(x_ref, h_ref, wx_ref, wh_ref, o_ref):
    r = (
        jnp.dot(x_ref[...], wx_ref[...], preferred_element_type=jnp.float32)
        + jnp.dot(h_ref[...], wh_ref[...], preferred_element_type=jnp.float32)
    )
    for k in range(4):
        o_ref[k] = r[:, k * 128:(k + 1) * 128]


def _matmul(X, H, Wx, Wh):
    return pl.pallas_call(
        _mm_body,
        grid=(10,),
        in_specs=[
            pl.BlockSpec((1000, 128), lambda i: (i, 0)),
            pl.BlockSpec((1000, 128), lambda i: (i, 0)),
            pl.BlockSpec((128, DG), lambda i: (0, 0)),
            pl.BlockSpec((128, DG), lambda i: (0, 0)),
        ],
        out_specs=pl.BlockSpec((4, 1000, 128), lambda i: (0, i, 0)),
        out_shape=jax.ShapeDtypeStruct((4, N, 128), jnp.float32),
    )(X, H, Wx, Wh)


def _sc_body(ep_hbm, m_hbm, z_hbm,
             s_out, dis_out,
             ssp, degsp,
             ebuf0, ebuf1, idx0, idx1, ab0, ab1, cb0, cb1,
             dis_t, gbuf0, gbuf1, tmp_t,
             esem0, esem1, gsem0, gsem1, ssem0, ssem1):
    c = lax.axis_index("c")
    s = lax.axis_index("s")
    base = s * RPT

    ebufs = (ebuf0, ebuf1)
    idxs = (idx0, idx1)
    abs_ = (ab0, ab1)
    cbs = (cb0, cb1)
    gbufs = (gbuf0, gbuf1)
    esems = (esem0, esem1)
    gsems = (gsem0, gsem1)
    ssems = (ssem0, ssem1)

    def estart(b, p):
        pltpu.async_copy(ep_hbm.at[s, b], ebufs[p], esems[p])

    def ewait(b, p):
        pltpu.make_async_copy(ep_hbm.at[s, b], ebufs[p], esems[p]).wait()

    def edrain():
        ewait(NBLK, 0)
        ewait(NBLK + 1, 1)

    def fill_ones(i, carry):
        tmp_t[pl.ds(i * 16, 16)] = jnp.ones((16,), jnp.float32)
        return carry
    lax.fori_loop(0, RPT // 16, fill_ones, 0)
    pltpu.sync_copy(tmp_t, degsp.at[pl.ds(base, RPT)])
    plsc.subcore_barrier()

    def deg_step(b, p, first):
        if not first:
            pltpu.make_async_copy(
                abs_[p], degsp.at[cbs[p]], ssems[p]).wait()
        ewait(b, p)

        def cvt(g, carry2):
            sl = pl.ds(g * 16, 16)
            abs_[p][sl] = lax.bitcast_convert_type(
                ebufs[p][2, sl], jnp.float32)
            cbs[p][sl] = ebufs[p][1, sl]
            return carry2
        lax.fori_loop(0, B // 16, cvt, 0)
        estart(b + 2, p)
        pltpu.async_copy(abs_[p], degsp.at[cbs[p]], ssems[p], add=True)

    estart(0, 0)
    estart(1, 1)
    deg_step(0, 0, True)
    deg_step(1, 1, True)

    def deg_pipe(j, carry):
        deg_step(2 * j + 2, 0, False)
        deg_step(2 * j + 3, 1, False)
        return carry
    lax.fori_loop(0, NBLK // 2 - 1, deg_pipe, 0)
    for p in range(2):
        pltpu.make_async_copy(abs_[p], degsp.at[cbs[p]], ssems[p]).wait()
    edrain()
    plsc.subcore_barrier()

    pltpu.sync_copy(degsp.at[pl.ds(base, RPT)], tmp_t)

    def rsqrt_vec(i, carry):
        x = tmp_t[pl.ds(i * 16, 16)]
        ii = lax.bitcast_convert_type(x, jnp.int32)
        ii = jnp.int32(0x5F3759DF) - lax.shift_right_logical(ii, 1)
        y = lax.bitcast_convert_type(ii, jnp.float32)
        y = y * (1.5 - 0.5 * x * y * y)
        y = y * (1.5 - 0.5 * x * y * y)
        y = y * (1.5 - 0.5 * x * y * y)
        tmp_t[pl.ds(i * 16, 16)] = y
        return carry
    lax.fori_loop(0, RPT // 16, rsqrt_vec, 0)
    pltpu.sync_copy(tmp_t, degsp.at[pl.ds(base, RPT)])

    @pl.when(c == 0)
    def _():
        pltpu.sync_copy(tmp_t, dis_out.at[pl.ds(base, RPT)])
    plsc.subcore_barrier()

    pltpu.sync_copy(degsp, dis_t)

    def chunk(k, carry):
        ci = c * NCH + k

        pltpu.sync_copy(z_hbm.at[pl.ds(base, RPT)], ssp.at[pl.ds(base, RPT)])
        plsc.subcore_barrier()

        def issue(b, p, first):
            if not first:
                for h in range(2):
                    pltpu.make_async_copy(
                        gbufs[p].at[pl.ds(h * (B // 2), B // 2)],
                        ssp.at[cbs[p].at[pl.ds(h * (B // 2), B // 2)]],
                        ssems[p]).wait()
            ewait(b, p)

            def prep(g, carry3):
                sl = pl.ds(g * 16, 16)
                r16 = ebufs[p][0, sl]
                idxs[p][sl] = r16 + ci * N
                cbs[p][sl] = ebufs[p][1, sl]
                ew16 = lax.bitcast_convert_type(ebufs[p][2, sl], jnp.float32)
                abs_[p][sl] = ew16 * plsc.load_gather(dis_t, [r16])
                return carry3
            lax.fori_loop(0, B // 16, prep, 0)
            estart(b + 2, p)
            for h in range(2):
                pltpu.async_copy(
                    m_hbm.at[idxs[p].at[pl.ds(h * (B // 2), B // 2)]],
                    gbufs[p].at[pl.ds(h * (B // 2), B // 2)], gsems[p])

        def retire(b, p):
            def gwait(h):
                pltpu.make_async_copy(
                    m_hbm.at[idxs[p].at[pl.ds(h * (B // 2), B // 2)]],
                    gbufs[p].at[pl.ds(h * (B // 2), B // 2)], gsems[p]).wait()

            def grp(g, carry3):
                a16 = abs_[p][pl.ds(g * 16, 16)]
                for e in range(16):
                    ae = jnp.take_along_axis(
                        a16, jnp.full((16,), e, jnp.int32), axis=0)
                    for q in range(W // 16):
                        v = gbufs[p][g * 16 + e, pl.ds(q * 16, 16)]
                        gbufs[p][g * 16 + e, pl.ds(q * 16, 16)] = v * ae
                return carry3
            gwait(0)
            lax.fori_loop(0, B // 32, grp, 0)
            pltpu.async_copy(gbufs[p].at[pl.ds(0, B // 2)],
                             ssp.at[cbs[p].at[pl.ds(0, B // 2)]],
                             ssems[p], add=True)
            gwait(1)
            lax.fori_loop(B // 32, B // 16, grp, 0)
            pltpu.async_copy(gbufs[p].at[pl.ds(B // 2, B // 2)],
                             ssp.at[cbs[p].at[pl.ds(B // 2, B // 2)]],
                             ssems[p], add=True)

        estart(0, 0)
        estart(1, 1)
        issue(0, 0, True)
        issue(1, 1, True)

        def pipe(j, carry2):
            retire(2 * j, 0)
            issue(2 * j + 2, 0, False)
            retire(2 * j + 1, 1)
            issue(2 * j + 3, 1, False)
            return carry2
        lax.fori_loop(0, NBLK // 2 - 1, pipe, 0)
        retire(NBLK - 2, 0)
        retire(NBLK - 1, 1)
        for p in range(2):
            for h in range(2):
                pltpu.make_async_copy(
                    gbufs[p].at[pl.ds(h * (B // 2), B // 2)],
                    ssp.at[cbs[p].at[pl.ds(h * (B // 2), B // 2)]],
                    ssems[p]).wait()
        edrain()

        plsc.subcore_barrier()
        pltpu.sync_copy(ssp.at[pl.ds(base, RPT)],
                        s_out.at[ci, pl.ds(base, RPT)])
        plsc.subcore_barrier()
        return carry
    lax.fori_loop(0, NCH, chunk, 0)


def _sc_call(epack, m_flat, zeros):
    mesh = plsc.VectorSubcoreMesh(core_axis_name="c", subcore_axis_name="s",
                                  num_cores=2, num_subcores=NS)
    return pl.kernel(
        _sc_body,
        out_type=(
            jax.ShapeDtypeStruct((2 * NCH, NPAD, W), jnp.float32),
            jax.ShapeDtypeStruct((NPAD,), jnp.float32),
        ),
        mesh=mesh,
        compiler_params=pltpu.CompilerParams(needs_layout_passes=False, use_tc_tiling_on_sc=False),
        scratch_types=[
            pltpu.VMEM_SHARED((NPAD, W), jnp.float32),
            pltpu.VMEM_SHARED((NPAD,), jnp.float32),
            pltpu.VMEM((3, B), jnp.int32),
            pltpu.VMEM((3, B), jnp.int32),
            pltpu.VMEM((B,), jnp.int32),
            pltpu.VMEM((B,), jnp.int32),
            pltpu.VMEM((B,), jnp.float32),
            pltpu.VMEM((B,), jnp.float32),
            pltpu.VMEM((B,), jnp.int32),
            pltpu.VMEM((B,), jnp.int32),
            pltpu.VMEM((NPAD,), jnp.float32),
            pltpu.VMEM((B, W), jnp.float32),
            pltpu.VMEM((B, W), jnp.float32),
            pltpu.VMEM((RPT,), jnp.float32),
            pltpu.SemaphoreType.DMA,
            pltpu.SemaphoreType.DMA,
            pltpu.SemaphoreType.DMA,
            pltpu.SemaphoreType.DMA,
            pltpu.SemaphoreType.DMA,
            pltpu.SemaphoreType.DMA,
        ],
    )(epack, m_flat, zeros)


def _gate_body(s_ref, m_ref, c_ref, dis_ref, bi, bf, bc, bo, wci, wcf, wco,
               h_out, c_out):
    dis = dis_ref[...]
    dis2 = dis * dis
    Cc = c_ref[...]
    Pi = dis * s_ref[0] + dis2 * m_ref[0]
    Pf = dis * s_ref[1] + dis2 * m_ref[1]
    Pc = dis * s_ref[2] + dis2 * m_ref[2]
    Po = dis * s_ref[3] + dis2 * m_ref[3]
    I = jax.nn.sigmoid(Pi + bi[...] + wci[...] * Cc)
    F = jax.nn.sigmoid(Pf + bf[...] + wcf[...] * Cc)
    T = jnp.tanh(Pc + bc[...])
    Cn = F * Cc + I * T
    O = jax.nn.sigmoid(Po + bo[...] + wco[...] * Cn)
    h_out[...] = O * jnp.tanh(Cn)
    c_out[...] = Cn


def _gating(s_ch, M4, C, dis_pad, bi, bf, bc, bo, wci, wcf, wco):
    vec = lambda: pl.BlockSpec((1, 128), lambda i: (0, 0))
    return pl.pallas_call(
        _gate_body,
        grid=(10,),
        in_specs=[
            pl.BlockSpec((4, 1000, 128), lambda i: (0, i, 0)),
            pl.BlockSpec((4, 1000, 128), lambda i: (0, i, 0)),
            pl.BlockSpec((1000, 128), lambda i: (i, 0)),
            pl.BlockSpec((1000, 1), lambda i: (i, 0)),
            vec(), vec(), vec(), vec(), vec(), vec(), vec(),
        ],
        out_specs=[
            pl.BlockSpec((1000, 128), lambda i: (i, 0)),
            pl.BlockSpec((1000, 128), lambda i: (i, 0)),
        ],
        out_shape=[
            jax.ShapeDtypeStruct((N, 128), jnp.float32),
            jax.ShapeDtypeStruct((N, 128), jnp.float32),
        ],
    )(s_ch, M4, C, dis_pad, bi, bf, bc, bo, wci, wcf, wco)


def kernel(X, edge_index, edge_weight, H, C,
           W_x_i, b_x_i, W_h_i, b_h_i,
           W_x_f, b_x_f, W_h_f, b_h_f,
           W_x_c, b_x_c, W_h_c, b_h_c,
           W_x_o, b_x_o, W_h_o, b_h_o,
           w_c_i, w_c_f, w_c_o, b_i, b_f, b_c, b_o):
    ei = edge_index.astype(jnp.int32)
    pad_i = jnp.zeros((EPAD - E,), jnp.int32)
    row_p = jnp.concatenate([ei[0], pad_i]).reshape(NS, NBLK, 1, B)
    col_p = jnp.concatenate([ei[1], pad_i]).reshape(NS, NBLK, 1, B)
    ew_p = jnp.concatenate(
        [lax.bitcast_convert_type(edge_weight, jnp.int32), pad_i]
    ).reshape(NS, NBLK, 1, B)
    epack = jnp.concatenate([row_p, col_p, ew_p], axis=2)
    epack = jnp.concatenate(
        [epack, jnp.zeros((NS, 2, 3, B), jnp.int32)], axis=1)

    Wx = jnp.concatenate([W_x_i, W_x_f, W_x_c, W_x_o], axis=1)
    Wh = jnp.concatenate([W_h_i, W_h_f, W_h_c, W_h_o], axis=1)
    M4 = _matmul(X, H, Wx, Wh)

    m_flat = M4.reshape(2 * NCH * N, W)
    zeros = jnp.zeros((NPAD, W), jnp.float32)

    s_ch, dis_pad = _sc_call(epack, m_flat, zeros)
    dis2d = dis_pad.reshape(NPAD, 1)

    bi = (b_x_i + b_h_i).reshape(1, 128) + b_i
    bf = (b_x_f + b_h_f).reshape(1, 128) + b_f
    bc = (b_x_c + b_h_c).reshape(1, 128) + b_c
    bo = (b_x_o + b_h_o).reshape(1, 128) + b_o

    H_new, C_new = _gating(s_ch, M4, C, dis2d, bi, bf, bc, bo,
                           w_c_i, w_c_f, w_c_o)
    return (H_new, C_new)
